# baseline (speedup 1.0000x reference)
import functools

import jax
import jax.numpy as jnp
from jax.experimental import pallas as pl
from jax.experimental.pallas import tpu as pltpu

HIDDEN = 1024
INTER = 1024
ALPHA = 1.702
LIMIT = 7.0
NC = 2
CW = 2 * INTER // NC
IW = INTER // NC


def _experts_kernel(hs_ref, wgu_ref, wd_ref, bgu_ref, bd_ref, rw_ref,
                    rwf_ref, out_ref, hs_bf, wgu_s, wd_s, p_s,
                    *, bt: int, nt: int):
    e = pl.program_id(0)
    c = pl.program_id(1)
    t = pl.program_id(2)
    first_ec = jnp.logical_and(e == 0, c == 0)

    @pl.when(jnp.logical_and(first_ec, t == 0))
    def _build_perm():
        k = jax.lax.broadcasted_iota(jnp.int32, (CW, CW), 0)
        j = jax.lax.broadcasted_iota(jnp.int32, (CW, CW), 1)
        src = jnp.where(j < IW, 2 * j, 2 * (j - IW) + 1)
        p_s[...] = (k == src).astype(jnp.bfloat16)

    @pl.when(first_ec)
    def _cast_tokens():
        hs_bf[pl.ds(t * bt, bt), :] = hs_ref[...].astype(jnp.bfloat16)

    @pl.when(t == 0)
    def _prep_weights():
        wgu_s[...] = jnp.dot(wgu_ref[0].astype(jnp.bfloat16), p_s[...],
                             preferred_element_type=jnp.float32
                             ).astype(jnp.bfloat16)
        wd_s[...] = wd_ref[0].astype(jnp.bfloat16)

    hb = bt // 2

    def _half(base):
        x = hs_bf[pl.ds(base, hb), :]
        gu = jnp.dot(x, wgu_s[...], preferred_element_type=jnp.float32)
        gu = (gu + bgu_ref[0, 0]).astype(jnp.bfloat16)
        gate = gu[:, :IW]
        up = gu[:, IW:]
        gate = jnp.minimum(gate, jnp.bfloat16(LIMIT))
        up = jnp.clip(up, jnp.bfloat16(-LIMIT), jnp.bfloat16(LIMIT))
        glu = gate * jax.nn.sigmoid(gate * jnp.bfloat16(ALPHA))
        rws = rw_ref[0, pl.ds(base - t * bt, hb), :].astype(jnp.bfloat16)
        act = (up + jnp.bfloat16(1.0)) * glu * rws
        return jnp.dot(act, wd_s[...], preferred_element_type=jnp.float32)

    nxt0 = _half(t * bt)
    nxt1 = _half(t * bt + hb)
    sl0 = pl.ds(t * bt, hb)
    sl1 = pl.ds(t * bt + hb, hb)

    @pl.when(first_ec)
    def _init():
        bias = jnp.dot(rwf_ref[...], bd_ref[...],
                       preferred_element_type=jnp.float32)
        out_ref[sl0, :] = nxt0 + bias[:hb, :]
        out_ref[sl1, :] = nxt1 + bias[hb:, :]

    @pl.when(jnp.logical_not(first_ec))
    def _acc():
        out_ref[sl0, :] = out_ref[sl0, :] + nxt0
        out_ref[sl1, :] = out_ref[sl1, :] + nxt1


def kernel(hidden_states, router_indices, routing_weights, gate_up_proj,
           gate_up_proj_bias, down_proj, down_proj_bias):
    del router_indices
    T, H = hidden_states.shape
    E = routing_weights.shape[1]
    BT = 1024
    NT = T // BT

    b3 = gate_up_proj_bias.reshape(E, NC, 1, CW)
    bgu = jnp.concatenate([b3[..., 0::2], b3[..., 1::2]], axis=-1)
    rw = routing_weights.T.reshape(E, T, 1)

    def hs_idx(e, c, t):
        first_ec = jnp.logical_and(e == 0, c == 0)
        return (jnp.where(first_ec, t, NT - 1), 0)

    grid = (E, NC, NT)
    out = pl.pallas_call(
        functools.partial(_experts_kernel, bt=BT, nt=NT),
        grid=grid,
        in_specs=[
            pl.BlockSpec((BT, H), hs_idx),
            pl.BlockSpec((1, H, CW), lambda e, c, t: (e, 0, c)),
            pl.BlockSpec((1, IW, H), lambda e, c, t: (e, c, 0)),
            pl.BlockSpec((1, 1, 1, CW), lambda e, c, t: (e, c, 0, 0)),
            pl.BlockSpec((E, H), lambda e, c, t: (0, 0)),
            pl.BlockSpec((1, BT, 1), lambda e, c, t: (e, t, 0)),
            pl.BlockSpec((BT, E), lambda e, c, t: (t, 0)),
        ],
        out_specs=pl.BlockSpec((T, H), lambda e, c, t: (0, 0)),
        out_shape=jax.ShapeDtypeStruct((T, H), jnp.float32),
        scratch_shapes=[
            pltpu.VMEM((T, HIDDEN), jnp.bfloat16),
            pltpu.VMEM((HIDDEN, CW), jnp.bfloat16),
            pltpu.VMEM((IW, HIDDEN), jnp.bfloat16),
            pltpu.VMEM((CW, CW), jnp.bfloat16),
        ],
        compiler_params=pltpu.CompilerParams(
            dimension_semantics=("arbitrary", "arbitrary", "arbitrary"),
            vmem_limit_bytes=64 * 1024 * 1024,
        ),
    )(hidden_states, gate_up_proj, down_proj, bgu, down_proj_bias, rw,
      routing_weights)
    return out


# speedup vs baseline: 1.0050x; 1.0050x over previous
import functools

import jax
import jax.numpy as jnp
from jax.experimental import pallas as pl
from jax.experimental.pallas import tpu as pltpu

HIDDEN = 1024
INTER = 1024
ALPHA = 1.702
LIMIT = 7.0
NC = 2
CW = 2 * INTER // NC
IW = INTER // NC


def _experts_kernel(hs_ref, wgu_ref, wd_ref, bgu_ref, bd_ref, rw_ref,
                    rwf_ref, out_ref, hs_bf, wgu_s, wd_s, p_s,
                    *, bt: int, nt: int):
    e = pl.program_id(0)
    c = pl.program_id(1)
    t = pl.program_id(2)
    first_ec = jnp.logical_and(e == 0, c == 0)

    @pl.when(jnp.logical_and(first_ec, t == 0))
    def _build_perm():
        k = jax.lax.broadcasted_iota(jnp.int32, (CW, CW), 0)
        j = jax.lax.broadcasted_iota(jnp.int32, (CW, CW), 1)
        src = jnp.where(j < IW, 2 * j, 2 * (j - IW) + 1)
        p_s[...] = (k == src).astype(jnp.bfloat16)

    @pl.when(first_ec)
    def _cast_tokens():
        hs_bf[pl.ds(t * bt, bt), :] = hs_ref[...].astype(jnp.bfloat16)

    @pl.when(t == 0)
    def _prep_weights():
        wgu_s[...] = jnp.dot(wgu_ref[0].astype(jnp.bfloat16), p_s[...],
                             preferred_element_type=jnp.float32
                             ).astype(jnp.bfloat16)
        wd_s[...] = wd_ref[0].astype(jnp.bfloat16)

    sl = pl.ds(t * bt, bt)
    x = hs_bf[sl, :]
    gu = jnp.dot(x, wgu_s[...], preferred_element_type=jnp.float32)
    gu = (gu + bgu_ref[0, 0]).astype(jnp.bfloat16)
    gate = gu[:, :IW]
    up = gu[:, IW:]
    gate = jnp.minimum(gate, jnp.bfloat16(LIMIT))
    up = jnp.clip(up, jnp.bfloat16(-LIMIT), jnp.bfloat16(LIMIT))
    glu = gate * jax.nn.sigmoid(gate * jnp.bfloat16(ALPHA))
    act = (up + jnp.bfloat16(1.0)) * glu * rw_ref[0].astype(jnp.bfloat16)
    nxt = jnp.dot(act, wd_s[...], preferred_element_type=jnp.float32)

    @pl.when(first_ec)
    def _init():
        bias = jnp.dot(rwf_ref[...], bd_ref[...],
                       preferred_element_type=jnp.float32)
        out_ref[sl, :] = nxt + bias

    @pl.when(jnp.logical_not(first_ec))
    def _acc():
        out_ref[sl, :] = out_ref[sl, :] + nxt


def kernel(hidden_states, router_indices, routing_weights, gate_up_proj,
           gate_up_proj_bias, down_proj, down_proj_bias):
    del router_indices
    T, H = hidden_states.shape
    E = routing_weights.shape[1]
    BT = 1024
    NT = T // BT

    b3 = gate_up_proj_bias.reshape(E, NC, 1, CW)
    bgu = jnp.concatenate([b3[..., 0::2], b3[..., 1::2]], axis=-1)
    rw = routing_weights.T.reshape(E, T, 1)

    def hs_idx(e, c, t):
        first_ec = jnp.logical_and(e == 0, c == 0)
        return (jnp.where(first_ec, t, NT - 1), 0)

    grid = (E, NC, NT)
    out = pl.pallas_call(
        functools.partial(_experts_kernel, bt=BT, nt=NT),
        grid=grid,
        in_specs=[
            pl.BlockSpec((BT, H), hs_idx),
            pl.BlockSpec((1, H, CW), lambda e, c, t: (e, 0, c)),
            pl.BlockSpec((1, IW, H), lambda e, c, t: (e, c, 0)),
            pl.BlockSpec((1, 1, 1, CW), lambda e, c, t: (e, c, 0, 0)),
            pl.BlockSpec((E, H), lambda e, c, t: (0, 0)),
            pl.BlockSpec((1, BT, 1), lambda e, c, t: (e, t, 0)),
            pl.BlockSpec((BT, E), lambda e, c, t: (t, 0)),
        ],
        out_specs=pl.BlockSpec((T, H), lambda e, c, t: (0, 0)),
        out_shape=jax.ShapeDtypeStruct((T, H), jnp.float32),
        scratch_shapes=[
            pltpu.VMEM((T, HIDDEN), jnp.bfloat16),
            pltpu.VMEM((HIDDEN, CW), jnp.bfloat16),
            pltpu.VMEM((IW, HIDDEN), jnp.bfloat16),
            pltpu.VMEM((CW, CW), jnp.bfloat16),
        ],
        compiler_params=pltpu.CompilerParams(
            dimension_semantics=("arbitrary", "arbitrary", "arbitrary"),
            vmem_limit_bytes=64 * 1024 * 1024,
        ),
    )(hidden_states, gate_up_proj, down_proj, bgu, down_proj_bias, rw,
      routing_weights)
    return out
